# baseline (speedup 1.0000x reference)
import functools

import jax
import jax.numpy as jnp
from jax import lax
from jax.experimental import pallas as pl
from jax.experimental.pallas import tpu as pltpu
from jax.experimental.pallas import tpu_sc as plsc

N = 10000
E = 320000
D = 128
NC = 2
NS = 16
NW = NC * NS
NP = 10112
RPW = NP // NS
EROWS = 2560
RPWK = EROWS // NW
PH = 2
RPP = RPWK // PH

_mesh = plsc.VectorSubcoreMesh(
    core_axis_name="c", subcore_axis_name="s", num_cores=NC, num_subcores=NS
)


def _wid():
    return lax.axis_index("s") * NC + lax.axis_index("c")


@functools.partial(
    pl.kernel,
    out_type=jax.ShapeDtypeStruct((NC, NP, 16), jnp.float32),
    mesh=_mesh,
    scratch_types=[
        pltpu.VMEM((RPWK, 128), jnp.int32),
        pltpu.VMEM((128, 16), jnp.float32),
        pltpu.VMEM_SHARED((NP, 16), jnp.float32),
    ],
)
def _sc_deg(dst_hbm, ones_hbm, out_hbm, idx_d, ones_v, acc):
    c = lax.axis_index("c")
    s = lax.axis_index("s")
    wid = _wid()
    pltpu.sync_copy(dst_hbm.at[pl.ds(wid * RPWK, RPWK)], idx_d)
    pltpu.sync_copy(ones_hbm, ones_v)
    for i in range(RPW // 128):
        pltpu.sync_copy(ones_hbm, acc.at[pl.ds(s * RPW + i * 128, 128)])
    pltpu.sync_copy(
        ones_hbm.at[pl.ds(0, RPW % 128)],
        acc.at[pl.ds(s * RPW + (RPW // 128) * 128, RPW % 128)],
    )
    plsc.subcore_barrier()

    def body(k, carry):
        pltpu.sync_copy(ones_v, acc.at[idx_d.at[k]], add=True)
        return carry

    lax.fori_loop(0, RPWK, body, 0)
    plsc.subcore_barrier()
    pltpu.sync_copy(acc.at[pl.ds(s * RPW, RPW)], out_hbm.at[c, pl.ds(s * RPW, RPW)])


@functools.partial(
    pl.kernel,
    out_type=jax.ShapeDtypeStruct((NC, NP, D), jnp.float32),
    mesh=_mesh,
    scratch_types=[
        pltpu.VMEM((RPP, 128), jnp.int32),
        pltpu.VMEM((RPP, 128), jnp.int32),
        pltpu.VMEM((128, D), jnp.float32),
        pltpu.VMEM((128, D), jnp.float32),
        pltpu.VMEM_SHARED((NP, D), jnp.float32),
        pltpu.SemaphoreType.DMA,
        pltpu.SemaphoreType.DMA,
    ],
)
def _sc_prop(u_hbm, src_hbm, dst_hbm, out_hbm, idx_s, idx_d, rows0, rows1,
             acc, sem0, sem1):
    c = lax.axis_index("c")
    s = lax.axis_index("s")
    wid = _wid()
    pltpu.sync_copy(u_hbm.at[pl.ds(s * RPW, RPW)], acc.at[pl.ds(s * RPW, RPW)])
    plsc.subcore_barrier()

    for ph in range(PH):
        base = wid * RPWK + ph * RPP
        pltpu.sync_copy(src_hbm.at[pl.ds(base, RPP)], idx_s)
        pltpu.sync_copy(dst_hbm.at[pl.ds(base, RPP)], idx_d)
        pltpu.async_copy(u_hbm.at[idx_s.at[0]], rows0, sem0)
        pltpu.async_copy(u_hbm.at[idx_s.at[1]], rows1, sem1)

        def body(k2, carry):
            k = 2 * k2
            pltpu.make_async_copy(u_hbm.at[idx_s.at[0]], rows0, sem0).wait()
            pltpu.sync_copy(rows0, acc.at[idx_d.at[k]], add=True)
            pltpu.async_copy(
                u_hbm.at[idx_s.at[jnp.minimum(k + 2, RPP - 1)]], rows0, sem0)
            pltpu.make_async_copy(u_hbm.at[idx_s.at[0]], rows1, sem1).wait()
            pltpu.sync_copy(rows1, acc.at[idx_d.at[k + 1]], add=True)
            pltpu.async_copy(
                u_hbm.at[idx_s.at[jnp.minimum(k + 3, RPP - 1)]], rows1, sem1)
            return carry

        lax.fori_loop(0, RPP // 2, body, 0)
        pltpu.make_async_copy(u_hbm.at[idx_s.at[0]], rows0, sem0).wait()
        pltpu.make_async_copy(u_hbm.at[idx_s.at[0]], rows1, sem1).wait()
    plsc.subcore_barrier()
    pltpu.sync_copy(acc.at[pl.ds(s * RPW, RPW)], out_hbm.at[c, pl.ds(s * RPW, RPW)])


def _tc_pre_body(dega, degb, x, dinv_ref, u0_ref):
    deg = dega[:, 0:1] + degb[:, 0:1] - 1.0
    dinv = lax.rsqrt(deg)
    dinv_ref[...] = dinv
    u0_ref[...] = dinv * x[...]


def _tc_pre(dega, degb, x):
    return pl.pallas_call(
        _tc_pre_body,
        out_shape=[
            jax.ShapeDtypeStruct((NP, 1), jnp.float32),
            jax.ShapeDtypeStruct((NP, D), jnp.float32),
        ],
    )(dega, degb, x)


def _tc_mid_body(s0a, s0b, u0, dinv, w1, w2, b1, u1_ref):
    y = dinv[...] * (s0a[...] + s0b[...] - u0[...])
    w12 = jnp.dot(w1[...], w2[...], precision=lax.Precision.HIGHEST)
    c = jnp.dot(b1[...], w2[...], precision=lax.Precision.HIGHEST)
    z = jnp.dot(y, w12, precision=lax.Precision.HIGHEST) + c
    u1_ref[...] = dinv[...] * z


def _tc_mid(s0a, s0b, u0, dinv, w1, w2, b1):
    return pl.pallas_call(
        _tc_mid_body,
        out_shape=jax.ShapeDtypeStruct((NP, D), jnp.float32),
    )(s0a, s0b, u0, dinv, w1, w2, b1)


def _tc_post_body(s1a, s1b, u1, dinv, b2, gamma, beta, out_ref):
    h2 = dinv[...] * (s1a[...] + s1b[...] - u1[...]) + b2[...]
    row = lax.broadcasted_iota(jnp.int32, (NP, 1), 0)
    valid = (row < N).astype(jnp.float32)
    h2v = h2 * valid
    mean = jnp.sum(h2v, axis=0, keepdims=True) * (1.0 / N)
    cent = (h2 - mean) * valid
    var = jnp.sum(cent * cent, axis=0, keepdims=True) * (1.0 / N)
    out_ref[...] = (h2 - mean) * lax.rsqrt(var + 1e-5) * gamma[...] + beta[...]


def _tc_post(s1a, s1b, u1, dinv, b2, gamma, beta):
    return pl.pallas_call(
        _tc_post_body,
        out_shape=jax.ShapeDtypeStruct((NP, D), jnp.float32),
    )(s1a, s1b, u1, dinv, b2, gamma, beta)


def kernel(edge_index, emb_weight, W1, b1, W2, b2, bn_gamma, bn_beta):
    src = edge_index[0].astype(jnp.int32)
    dst = edge_index[1].astype(jnp.int32)
    ppw = (EROWS * 128 - E) // NW
    epw = E // NW
    dummy_src = jnp.zeros((NW, ppw), jnp.int32)
    dummy_dst = jnp.broadcast_to(
        N + (jnp.arange(ppw, dtype=jnp.int32) % (NP - N - 1)) + 1, (NW, ppw))
    srcp = jnp.concatenate(
        [src.reshape(NW, epw), dummy_src], axis=1).reshape(EROWS, 128)
    dstp = jnp.concatenate(
        [dst.reshape(NW, epw), dummy_dst], axis=1).reshape(EROWS, 128)

    ones128 = jnp.ones((128, 16), jnp.float32)
    xpad = jnp.zeros((NP, D), jnp.float32).at[:N].set(emb_weight)

    deg_parts = _sc_deg(dstp, ones128)
    dinv, u0 = _tc_pre(deg_parts[0], deg_parts[1], xpad)
    s0 = _sc_prop(u0, srcp, dstp)
    u1 = _tc_mid(s0[0], s0[1], u0, dinv, W1, W2, b1.reshape(1, -1))
    s1 = _sc_prop(u1, srcp, dstp)
    out = _tc_post(
        s1[0], s1[1], u1, dinv,
        b2.reshape(1, -1), bn_gamma.reshape(1, -1), bn_beta.reshape(1, -1),
    )
    return out[:N]


# speedup vs baseline: 1.4711x; 1.4711x over previous
import functools

import jax
import jax.numpy as jnp
from jax import lax
from jax.experimental import pallas as pl
from jax.experimental.pallas import tpu as pltpu
from jax.experimental.pallas import tpu_sc as plsc

N = 10000
E = 320000
D = 128
NC = 2
NS = 16
NW = NC * NS
NP = 10112
RPW = NP // NS
EROWS = 2560
RPWK = EROWS // NW
PH = 2
RPP = RPWK // PH

_mesh = plsc.VectorSubcoreMesh(
    core_axis_name="c", subcore_axis_name="s", num_cores=NC, num_subcores=NS
)


def _wid():
    return lax.axis_index("s") * NC + lax.axis_index("c")


@functools.partial(
    pl.kernel,
    out_type=jax.ShapeDtypeStruct((NC, NP, 16), jnp.float32),
    mesh=_mesh,
    scratch_types=[
        pltpu.VMEM((RPWK, 128), jnp.int32),
        pltpu.VMEM((128, 16), jnp.float32),
        pltpu.VMEM_SHARED((NP, 16), jnp.float32),
    ],
)
def _sc_deg(dst_hbm, ones_hbm, out_hbm, idx_d, ones_v, acc):
    c = lax.axis_index("c")
    s = lax.axis_index("s")
    wid = _wid()
    pltpu.sync_copy(dst_hbm.at[pl.ds(wid * RPWK, RPWK)], idx_d)
    pltpu.sync_copy(ones_hbm, ones_v)
    for i in range(RPW // 128):
        pltpu.sync_copy(ones_hbm, acc.at[pl.ds(s * RPW + i * 128, 128)])
    pltpu.sync_copy(
        ones_hbm.at[pl.ds(0, RPW % 128)],
        acc.at[pl.ds(s * RPW + (RPW // 128) * 128, RPW % 128)],
    )
    plsc.subcore_barrier()

    def body(k, carry):
        pltpu.sync_copy(ones_v, acc.at[idx_d.at[k]], add=True)
        return carry

    lax.fori_loop(0, RPWK, body, 0)
    plsc.subcore_barrier()
    pltpu.sync_copy(acc.at[pl.ds(s * RPW, RPW)], out_hbm.at[c, pl.ds(s * RPW, RPW)])


@functools.partial(
    pl.kernel,
    out_type=jax.ShapeDtypeStruct((NC, NP, D), jnp.float32),
    mesh=_mesh,
    scratch_types=[
        pltpu.VMEM((RPP, 128), jnp.int32),
        pltpu.VMEM((RPP, 128), jnp.int32),
        pltpu.VMEM((128, D), jnp.float32),
        pltpu.VMEM((128, D), jnp.float32),
        pltpu.VMEM_SHARED((NP, D), jnp.float32),
        pltpu.SemaphoreType.DMA,
        pltpu.SemaphoreType.DMA,
    ],
)
def _sc_prop(u_hbm, src_hbm, dst_hbm, out_hbm, idx_s, idx_d, rows0, rows1,
             acc, sem0, sem1):
    c = lax.axis_index("c")
    s = lax.axis_index("s")
    wid = _wid()
    pltpu.sync_copy(u_hbm.at[pl.ds(s * RPW, RPW)], acc.at[pl.ds(s * RPW, RPW)])
    plsc.subcore_barrier()

    for ph in range(PH):
        base = wid * RPWK + ph * RPP
        pltpu.sync_copy(src_hbm.at[pl.ds(base, RPP)], idx_s)
        pltpu.sync_copy(dst_hbm.at[pl.ds(base, RPP)], idx_d)

        def body(k, carry):
            pltpu.async_copy(u_hbm.at[idx_s.at[k]], rows0, sem0).wait()
            pltpu.sync_copy(rows0, acc.at[idx_d.at[k]], add=True)
            return carry

        lax.fori_loop(0, RPP, body, 0)
    plsc.subcore_barrier()
    pltpu.sync_copy(acc.at[pl.ds(s * RPW, RPW)], out_hbm.at[c, pl.ds(s * RPW, RPW)])


def _tc_pre_body(dega, degb, x, dinv_ref, u0_ref):
    deg = dega[:, 0:1] + degb[:, 0:1] - 1.0
    dinv = lax.rsqrt(deg)
    dinv_ref[...] = dinv
    u0_ref[...] = dinv * x[...]


def _tc_pre(dega, degb, x):
    return pl.pallas_call(
        _tc_pre_body,
        out_shape=[
            jax.ShapeDtypeStruct((NP, 1), jnp.float32),
            jax.ShapeDtypeStruct((NP, D), jnp.float32),
        ],
    )(dega, degb, x)


def _tc_mid_body(s0a, s0b, u0, dinv, w1, w2, b1, u1_ref):
    y = dinv[...] * (s0a[...] + s0b[...] - u0[...])
    w12 = jnp.dot(w1[...], w2[...], precision=lax.Precision.HIGHEST)
    c = jnp.dot(b1[...], w2[...], precision=lax.Precision.HIGHEST)
    z = jnp.dot(y, w12, precision=lax.Precision.HIGHEST) + c
    u1_ref[...] = dinv[...] * z


def _tc_mid(s0a, s0b, u0, dinv, w1, w2, b1):
    return pl.pallas_call(
        _tc_mid_body,
        out_shape=jax.ShapeDtypeStruct((NP, D), jnp.float32),
    )(s0a, s0b, u0, dinv, w1, w2, b1)


def _tc_post_body(s1a, s1b, u1, dinv, b2, gamma, beta, out_ref):
    h2 = dinv[...] * (s1a[...] + s1b[...] - u1[...]) + b2[...]
    row = lax.broadcasted_iota(jnp.int32, (NP, 1), 0)
    valid = (row < N).astype(jnp.float32)
    h2v = h2 * valid
    mean = jnp.sum(h2v, axis=0, keepdims=True) * (1.0 / N)
    cent = (h2 - mean) * valid
    var = jnp.sum(cent * cent, axis=0, keepdims=True) * (1.0 / N)
    out_ref[...] = (h2 - mean) * lax.rsqrt(var + 1e-5) * gamma[...] + beta[...]


def _tc_post(s1a, s1b, u1, dinv, b2, gamma, beta):
    return pl.pallas_call(
        _tc_post_body,
        out_shape=jax.ShapeDtypeStruct((NP, D), jnp.float32),
    )(s1a, s1b, u1, dinv, b2, gamma, beta)


def kernel(edge_index, emb_weight, W1, b1, W2, b2, bn_gamma, bn_beta):
    src = edge_index[0].astype(jnp.int32)
    dst = edge_index[1].astype(jnp.int32)
    ppw = (EROWS * 128 - E) // NW
    epw = E // NW
    dummy_src = jnp.zeros((NW, ppw), jnp.int32)
    dummy_dst = jnp.broadcast_to(
        N + (jnp.arange(ppw, dtype=jnp.int32) % (NP - N - 1)) + 1, (NW, ppw))
    srcp = jnp.concatenate(
        [src.reshape(NW, epw), dummy_src], axis=1).reshape(EROWS, 128)
    dstp = jnp.concatenate(
        [dst.reshape(NW, epw), dummy_dst], axis=1).reshape(EROWS, 128)

    ones128 = jnp.ones((128, 16), jnp.float32)
    xpad = jnp.zeros((NP, D), jnp.float32).at[:N].set(emb_weight)

    deg_parts = _sc_deg(dstp, ones128)
    dinv, u0 = _tc_pre(deg_parts[0], deg_parts[1], xpad)
    s0 = _sc_prop(u0, srcp, dstp)
    u1 = _tc_mid(s0[0], s0[1], u0, dinv, W1, W2, b1.reshape(1, -1))
    s1 = _sc_prop(u1, srcp, dstp)
    out = _tc_post(
        s1[0], s1[1], u1, dinv,
        b2.reshape(1, -1), bn_gamma.reshape(1, -1), bn_beta.reshape(1, -1),
    )
    return out[:N]


# speedup vs baseline: 3.1507x; 2.1418x over previous
import functools

import jax
import jax.numpy as jnp
from jax import lax
from jax.experimental import pallas as pl
from jax.experimental.pallas import tpu as pltpu
from jax.experimental.pallas import tpu_sc as plsc

N = 10000
E = 320000
D = 128
NC = 2
NS = 16
NW = NC * NS
NP = 10112
RPW = NP // NS
EPW = E // NW
KCH = 78
TAIL = EPW - KCH * 128

_mesh = plsc.VectorSubcoreMesh(
    core_axis_name="c", subcore_axis_name="s", num_cores=NC, num_subcores=NS
)


def _wid():
    return lax.axis_index("s") * NC + lax.axis_index("c")


@functools.partial(
    pl.kernel,
    out_type=jax.ShapeDtypeStruct((NC, NP, 16), jnp.float32),
    mesh=_mesh,
    scratch_types=[
        pltpu.VMEM((KCH, 128), jnp.int32),
        pltpu.VMEM((TAIL,), jnp.int32),
        pltpu.VMEM((128, 16), jnp.float32),
        pltpu.VMEM_SHARED((NP, 16), jnp.float32),
    ],
)
def _sc_deg(dst_hbm, dstt_hbm, ones_hbm, out_hbm, idx_d, tidx_d, ones_v, acc):
    c = lax.axis_index("c")
    s = lax.axis_index("s")
    wid = _wid()
    pltpu.sync_copy(dst_hbm.at[wid], idx_d)
    pltpu.sync_copy(dstt_hbm.at[wid], tidx_d)
    pltpu.sync_copy(ones_hbm, ones_v)
    for i in range(RPW // 128):
        pltpu.sync_copy(ones_hbm, acc.at[pl.ds(s * RPW + i * 128, 128)])
    pltpu.sync_copy(
        ones_hbm.at[pl.ds(0, RPW % 128)],
        acc.at[pl.ds(s * RPW + (RPW // 128) * 128, RPW % 128)],
    )
    plsc.subcore_barrier()

    def body(k, carry):
        pltpu.sync_copy(ones_v, acc.at[idx_d.at[k]], add=True)
        return carry

    lax.fori_loop(0, KCH, body, 0)
    pltpu.sync_copy(ones_v.at[pl.ds(0, TAIL)], acc.at[tidx_d], add=True)
    plsc.subcore_barrier()
    pltpu.sync_copy(acc.at[pl.ds(s * RPW, RPW)], out_hbm.at[c, pl.ds(s * RPW, RPW)])


@functools.partial(
    pl.kernel,
    out_type=jax.ShapeDtypeStruct((NC, NP, D), jnp.float32),
    mesh=_mesh,
    scratch_types=[
        pltpu.VMEM((KCH, 128), jnp.int32),
        pltpu.VMEM((KCH, 128), jnp.int32),
        pltpu.VMEM((TAIL,), jnp.int32),
        pltpu.VMEM((TAIL,), jnp.int32),
        pltpu.VMEM((128, D), jnp.float32),
        pltpu.VMEM((TAIL, D), jnp.float32),
        pltpu.VMEM_SHARED((NP, D), jnp.float32),
        pltpu.SemaphoreType.DMA,
    ],
)
def _sc_prop(u_hbm, src_hbm, dst_hbm, srct_hbm, dstt_hbm, out_hbm,
             idx_s, idx_d, tidx_s, tidx_d, rows, trows, acc, sem):
    c = lax.axis_index("c")
    s = lax.axis_index("s")
    wid = _wid()
    pltpu.sync_copy(src_hbm.at[wid], idx_s)
    pltpu.sync_copy(dst_hbm.at[wid], idx_d)
    pltpu.sync_copy(srct_hbm.at[wid], tidx_s)
    pltpu.sync_copy(dstt_hbm.at[wid], tidx_d)
    pltpu.sync_copy(u_hbm.at[pl.ds(s * RPW, RPW)], acc.at[pl.ds(s * RPW, RPW)])
    plsc.subcore_barrier()

    def body(k, carry):
        pltpu.async_copy(u_hbm.at[idx_s.at[k]], rows, sem).wait()
        pltpu.sync_copy(rows, acc.at[idx_d.at[k]], add=True)
        return carry

    lax.fori_loop(0, KCH, body, 0)
    pltpu.async_copy(u_hbm.at[tidx_s], trows, sem).wait()
    pltpu.sync_copy(trows, acc.at[tidx_d], add=True)
    plsc.subcore_barrier()
    pltpu.sync_copy(acc.at[pl.ds(s * RPW, RPW)], out_hbm.at[c, pl.ds(s * RPW, RPW)])


def _tc_pre_body(dega, degb, x, dinv_ref, u0_ref):
    deg = dega[:, 0:1] + degb[:, 0:1] - 1.0
    dinv = lax.rsqrt(deg)
    dinv_ref[...] = dinv
    u0_ref[...] = dinv * x[...]


def _tc_pre(dega, degb, x):
    return pl.pallas_call(
        _tc_pre_body,
        out_shape=[
            jax.ShapeDtypeStruct((NP, 1), jnp.float32),
            jax.ShapeDtypeStruct((NP, D), jnp.float32),
        ],
    )(dega, degb, x)


def _tc_mid_body(s0a, s0b, u0, dinv, w1, w2, b1, u1_ref):
    y = dinv[...] * (s0a[...] + s0b[...] - u0[...])
    w12 = jnp.dot(w1[...], w2[...], precision=lax.Precision.HIGHEST)
    c = jnp.dot(b1[...], w2[...], precision=lax.Precision.HIGHEST)
    z = jnp.dot(y, w12, precision=lax.Precision.HIGHEST) + c
    u1_ref[...] = dinv[...] * z


def _tc_mid(s0a, s0b, u0, dinv, w1, w2, b1):
    return pl.pallas_call(
        _tc_mid_body,
        out_shape=jax.ShapeDtypeStruct((NP, D), jnp.float32),
    )(s0a, s0b, u0, dinv, w1, w2, b1)


def _tc_post_body(s1a, s1b, u1, dinv, b2, gamma, beta, out_ref):
    h2 = dinv[...] * (s1a[...] + s1b[...] - u1[...]) + b2[...]
    row = lax.broadcasted_iota(jnp.int32, (NP, 1), 0)
    valid = (row < N).astype(jnp.float32)
    h2v = h2 * valid
    mean = jnp.sum(h2v, axis=0, keepdims=True) * (1.0 / N)
    cent = (h2 - mean) * valid
    var = jnp.sum(cent * cent, axis=0, keepdims=True) * (1.0 / N)
    out_ref[...] = (h2 - mean) * lax.rsqrt(var + 1e-5) * gamma[...] + beta[...]


def _tc_post(s1a, s1b, u1, dinv, b2, gamma, beta):
    return pl.pallas_call(
        _tc_post_body,
        out_shape=jax.ShapeDtypeStruct((NP, D), jnp.float32),
    )(s1a, s1b, u1, dinv, b2, gamma, beta)


def kernel(edge_index, emb_weight, W1, b1, W2, b2, bn_gamma, bn_beta):
    src = edge_index[0].astype(jnp.int32).reshape(NW, EPW)
    dst = edge_index[1].astype(jnp.int32).reshape(NW, EPW)
    src_main = src[:, : KCH * 128].reshape(NW, KCH, 128)
    dst_main = dst[:, : KCH * 128].reshape(NW, KCH, 128)
    src_tail = src[:, KCH * 128:]
    dst_tail = dst[:, KCH * 128:]

    ones128 = jnp.ones((128, 16), jnp.float32)
    xpad = jnp.zeros((NP, D), jnp.float32).at[:N].set(emb_weight)

    deg_parts = _sc_deg(dst_main, dst_tail, ones128)
    dinv, u0 = _tc_pre(deg_parts[0], deg_parts[1], xpad)
    s0 = _sc_prop(u0, src_main, dst_main, src_tail, dst_tail)
    u1 = _tc_mid(s0[0], s0[1], u0, dinv, W1, W2, b1.reshape(1, -1))
    s1 = _sc_prop(u1, src_main, dst_main, src_tail, dst_tail)
    out = _tc_post(
        s1[0], s1[1], u1, dinv,
        b2.reshape(1, -1), bn_gamma.reshape(1, -1), bn_beta.reshape(1, -1),
    )
    return out[:N]


# speedup vs baseline: 3.5002x; 1.1109x over previous
import functools

import jax
import jax.numpy as jnp
from jax import lax
from jax.experimental import pallas as pl
from jax.experimental.pallas import tpu as pltpu
from jax.experimental.pallas import tpu_sc as plsc

N = 10000
E = 320000
D = 128
NC = 2
NS = 16
NW = NC * NS
NP = 10112
RPW = NP // NS
EPW = E // NW
KCH = 78
PH = 3
KPP = KCH // PH
TAIL = EPW - KCH * 128

_mesh = plsc.VectorSubcoreMesh(
    core_axis_name="c", subcore_axis_name="s", num_cores=NC, num_subcores=NS
)


def _wid():
    return lax.axis_index("s") * NC + lax.axis_index("c")


@functools.partial(
    pl.kernel,
    out_type=jax.ShapeDtypeStruct((NC, NP, 16), jnp.float32),
    mesh=_mesh,
    scratch_types=[
        pltpu.VMEM((PH, KPP, 128), jnp.int32),
        pltpu.VMEM((TAIL,), jnp.int32),
        pltpu.VMEM((128, 16), jnp.float32),
        pltpu.VMEM_SHARED((NP, 16), jnp.float32),
    ],
)
def _sc_deg(dst_hbm, dstt_hbm, ones_hbm, out_hbm, idx_d, tidx_d, ones_v, acc):
    c = lax.axis_index("c")
    s = lax.axis_index("s")
    wid = _wid()
    pltpu.sync_copy(dst_hbm.at[wid], idx_d)
    pltpu.sync_copy(dstt_hbm.at[wid], tidx_d)
    pltpu.sync_copy(ones_hbm, ones_v)
    for i in range(RPW // 128):
        pltpu.sync_copy(ones_hbm, acc.at[pl.ds(s * RPW + i * 128, 128)])
    pltpu.sync_copy(
        ones_hbm.at[pl.ds(0, RPW % 128)],
        acc.at[pl.ds(s * RPW + (RPW // 128) * 128, RPW % 128)],
    )
    plsc.subcore_barrier()

    for ph in range(PH):
        def body(k, carry):
            pltpu.sync_copy(ones_v, acc.at[idx_d.at[ph, k]], add=True)
            return carry

        lax.fori_loop(0, KPP, body, 0)
    pltpu.sync_copy(ones_v.at[pl.ds(0, TAIL)], acc.at[tidx_d], add=True)
    plsc.subcore_barrier()
    pltpu.sync_copy(acc.at[pl.ds(s * RPW, RPW)], out_hbm.at[c, pl.ds(s * RPW, RPW)])


@functools.partial(
    pl.kernel,
    out_type=jax.ShapeDtypeStruct((NC, NP, D), jnp.float32),
    mesh=_mesh,
    scratch_types=[
        pltpu.VMEM((KPP, 128), jnp.int32),
        pltpu.VMEM((KPP, 128), jnp.int32),
        pltpu.VMEM((TAIL,), jnp.int32),
        pltpu.VMEM((TAIL,), jnp.int32),
        pltpu.VMEM((128, D), jnp.float32),
        pltpu.VMEM((128, D), jnp.float32),
        pltpu.VMEM((TAIL, D), jnp.float32),
        pltpu.VMEM_SHARED((NP, D), jnp.float32),
        pltpu.SemaphoreType.DMA,
        pltpu.SemaphoreType.DMA,
        pltpu.SemaphoreType.DMA,
        pltpu.SemaphoreType.DMA,
    ],
)
def _sc_prop(u_hbm, src_hbm, dst_hbm, srct_hbm, dstt_hbm, out_hbm,
             idx_s, idx_d, tidx_s, tidx_d, rows0, rows1, trows, acc,
             semg0, semg1, sems0, sems1):
    c = lax.axis_index("c")
    s = lax.axis_index("s")
    wid = _wid()
    pltpu.sync_copy(srct_hbm.at[wid], tidx_s)
    pltpu.sync_copy(dstt_hbm.at[wid], tidx_d)
    pltpu.sync_copy(u_hbm.at[pl.ds(s * RPW, RPW)], acc.at[pl.ds(s * RPW, RPW)])
    plsc.subcore_barrier()

    for ph in range(PH):
        pltpu.sync_copy(src_hbm.at[wid, ph], idx_s)
        pltpu.sync_copy(dst_hbm.at[wid, ph], idx_d)

        def body(k2, carry):
            k = 2 * k2
            g0 = pltpu.async_copy(u_hbm.at[idx_s.at[k]], rows0, semg0)
            g1 = pltpu.async_copy(u_hbm.at[idx_s.at[k + 1]], rows1, semg1)
            g0.wait()
            s0 = pltpu.async_copy(rows0, acc.at[idx_d.at[k]], sems0, add=True)
            g1.wait()
            s1 = pltpu.async_copy(rows1, acc.at[idx_d.at[k + 1]], sems1, add=True)
            s0.wait()
            s1.wait()
            return carry

        lax.fori_loop(0, KPP // 2, body, 0)

    pltpu.async_copy(u_hbm.at[tidx_s], trows, semg0).wait()
    pltpu.sync_copy(trows, acc.at[tidx_d], add=True)
    plsc.subcore_barrier()
    pltpu.sync_copy(acc.at[pl.ds(s * RPW, RPW)], out_hbm.at[c, pl.ds(s * RPW, RPW)])


def _tc_pre_body(dega, degb, x, dinv_ref, u0_ref):
    deg = dega[:, 0:1] + degb[:, 0:1] - 1.0
    dinv = lax.rsqrt(deg)
    dinv_ref[...] = dinv
    u0_ref[...] = dinv * x[...]


def _tc_pre(dega, degb, x):
    return pl.pallas_call(
        _tc_pre_body,
        out_shape=[
            jax.ShapeDtypeStruct((NP, 1), jnp.float32),
            jax.ShapeDtypeStruct((NP, D), jnp.float32),
        ],
    )(dega, degb, x)


def _tc_mid_body(s0a, s0b, u0, dinv, w1, w2, b1, u1_ref):
    y = dinv[...] * (s0a[...] + s0b[...] - u0[...])
    w12 = jnp.dot(w1[...], w2[...], precision=lax.Precision.HIGHEST)
    c = jnp.dot(b1[...], w2[...], precision=lax.Precision.HIGHEST)
    z = jnp.dot(y, w12, precision=lax.Precision.HIGHEST) + c
    u1_ref[...] = dinv[...] * z


def _tc_mid(s0a, s0b, u0, dinv, w1, w2, b1):
    return pl.pallas_call(
        _tc_mid_body,
        out_shape=jax.ShapeDtypeStruct((NP, D), jnp.float32),
    )(s0a, s0b, u0, dinv, w1, w2, b1)


def _tc_post_body(s1a, s1b, u1, dinv, b2, gamma, beta, out_ref):
    h2 = dinv[...] * (s1a[...] + s1b[...] - u1[...]) + b2[...]
    row = lax.broadcasted_iota(jnp.int32, (NP, 1), 0)
    valid = (row < N).astype(jnp.float32)
    h2v = h2 * valid
    mean = jnp.sum(h2v, axis=0, keepdims=True) * (1.0 / N)
    cent = (h2 - mean) * valid
    var = jnp.sum(cent * cent, axis=0, keepdims=True) * (1.0 / N)
    out_ref[...] = (h2 - mean) * lax.rsqrt(var + 1e-5) * gamma[...] + beta[...]


def _tc_post(s1a, s1b, u1, dinv, b2, gamma, beta):
    return pl.pallas_call(
        _tc_post_body,
        out_shape=jax.ShapeDtypeStruct((NP, D), jnp.float32),
    )(s1a, s1b, u1, dinv, b2, gamma, beta)


def kernel(edge_index, emb_weight, W1, b1, W2, b2, bn_gamma, bn_beta):
    src = edge_index[0].astype(jnp.int32).reshape(NW, EPW)
    dst = edge_index[1].astype(jnp.int32).reshape(NW, EPW)
    src_main = src[:, : KCH * 128].reshape(NW, PH, KPP, 128)
    dst_main = dst[:, : KCH * 128].reshape(NW, PH, KPP, 128)
    src_tail = src[:, KCH * 128:]
    dst_tail = dst[:, KCH * 128:]

    ones128 = jnp.ones((128, 16), jnp.float32)
    xpad = jnp.zeros((NP, D), jnp.float32).at[:N].set(emb_weight)

    deg_parts = _sc_deg(dst_main, dst_tail, ones128)
    dinv, u0 = _tc_pre(deg_parts[0], deg_parts[1], xpad)
    s0 = _sc_prop(u0, src_main, dst_main, src_tail, dst_tail)
    u1 = _tc_mid(s0[0], s0[1], u0, dinv, W1, W2, b1.reshape(1, -1))
    s1 = _sc_prop(u1, src_main, dst_main, src_tail, dst_tail)
    out = _tc_post(
        s1[0], s1[1], u1, dinv,
        b2.reshape(1, -1), bn_gamma.reshape(1, -1), bn_beta.reshape(1, -1),
    )
    return out[:N]


# speedup vs baseline: 4.2112x; 1.2031x over previous
import functools

import jax
import jax.numpy as jnp
from jax import lax
from jax.experimental import pallas as pl
from jax.experimental.pallas import tpu as pltpu
from jax.experimental.pallas import tpu_sc as plsc

N = 10000
E = 320000
D = 128
NC = 2
NS = 16
NW = NC * NS
NP = 10112
RPW = NP // NS
EPW = E // NW
KCH = 78
PH = 3
KPP = KCH // PH
TAIL = EPW - KCH * 128

_mesh = plsc.VectorSubcoreMesh(
    core_axis_name="c", subcore_axis_name="s", num_cores=NC, num_subcores=NS
)


def _wid():
    return lax.axis_index("s") * NC + lax.axis_index("c")


@functools.partial(
    pl.kernel,
    out_type=jax.ShapeDtypeStruct((NC, NP, 16), jnp.float32),
    mesh=_mesh,
    scratch_types=[
        pltpu.VMEM((PH, KPP, 128), jnp.int32),
        pltpu.VMEM((TAIL,), jnp.int32),
        pltpu.VMEM((128, 16), jnp.float32),
        pltpu.VMEM_SHARED((NP, 16), jnp.float32),
        pltpu.SemaphoreType.DMA,
    ],
)
def _sc_deg(dst_hbm, dstt_hbm, ones_hbm, out_hbm, idx_d, tidx_d, ones_v, acc,
            sem):
    c = lax.axis_index("c")
    s = lax.axis_index("s")
    wid = _wid()
    pltpu.sync_copy(dst_hbm.at[wid], idx_d)
    pltpu.sync_copy(dstt_hbm.at[wid], tidx_d)
    pltpu.sync_copy(ones_hbm, ones_v)
    for i in range(RPW // 128):
        pltpu.sync_copy(ones_hbm, acc.at[pl.ds(s * RPW + i * 128, 128)])
    pltpu.sync_copy(
        ones_hbm.at[pl.ds(0, RPW % 128)],
        acc.at[pl.ds(s * RPW + (RPW // 128) * 128, RPW % 128)],
    )
    plsc.subcore_barrier()

    descs = []
    for ph in range(PH):
        for k in range(KPP):
            descs.append(pltpu.async_copy(
                ones_v, acc.at[idx_d.at[ph, k]], sem, add=True))
    for d in descs:
        d.wait()
    pltpu.sync_copy(ones_v.at[pl.ds(0, TAIL)], acc.at[tidx_d], add=True)
    plsc.subcore_barrier()
    pltpu.sync_copy(acc.at[pl.ds(s * RPW, RPW)], out_hbm.at[c, pl.ds(s * RPW, RPW)])


@functools.partial(
    pl.kernel,
    out_type=jax.ShapeDtypeStruct((NC, NP, D), jnp.float32),
    mesh=_mesh,
    scratch_types=[
        pltpu.VMEM((KPP, 128), jnp.int32),
        pltpu.VMEM((KPP, 128), jnp.int32),
        pltpu.VMEM((TAIL,), jnp.int32),
        pltpu.VMEM((TAIL,), jnp.int32),
        pltpu.VMEM((128, D), jnp.float32),
        pltpu.VMEM((128, D), jnp.float32),
        pltpu.VMEM((TAIL, D), jnp.float32),
        pltpu.VMEM_SHARED((NP, D), jnp.float32),
        pltpu.SemaphoreType.DMA,
        pltpu.SemaphoreType.DMA,
        pltpu.SemaphoreType.DMA,
        pltpu.SemaphoreType.DMA,
    ],
)
def _sc_prop(u_hbm, src_hbm, dst_hbm, srct_hbm, dstt_hbm, out_hbm,
             idx_s, idx_d, tidx_s, tidx_d, rows0, rows1, trows, acc,
             semg0, semg1, sems0, sems1):
    c = lax.axis_index("c")
    s = lax.axis_index("s")
    wid = _wid()
    pltpu.sync_copy(srct_hbm.at[wid], tidx_s)
    pltpu.sync_copy(dstt_hbm.at[wid], tidx_d)
    pltpu.sync_copy(u_hbm.at[pl.ds(s * RPW, RPW)], acc.at[pl.ds(s * RPW, RPW)])
    plsc.subcore_barrier()

    bufs = (rows0, rows1)
    gsems = (semg0, semg1)
    ssems = (sems0, sems1)
    for ph in range(PH):
        pltpu.sync_copy(src_hbm.at[wid, ph], idx_s)
        pltpu.sync_copy(dst_hbm.at[wid, ph], idx_d)
        gd = [None, None]
        sd = [None, None]
        for k in range(KPP):
            p = k & 1
            if sd[p] is not None:
                sd[p].wait()
            gd[p] = pltpu.async_copy(u_hbm.at[idx_s.at[k]], bufs[p], gsems[p])
            if k > 0:
                q = (k - 1) & 1
                gd[q].wait()
                sd[q] = pltpu.async_copy(
                    bufs[q], acc.at[idx_d.at[k - 1]], ssems[q], add=True)
        p = (KPP - 1) & 1
        gd[p].wait()
        sd[p] = pltpu.async_copy(
            bufs[p], acc.at[idx_d.at[KPP - 1]], ssems[p], add=True)
        sd[0].wait()
        sd[1].wait()

    pltpu.async_copy(u_hbm.at[tidx_s], trows, semg0).wait()
    pltpu.sync_copy(trows, acc.at[tidx_d], add=True)
    plsc.subcore_barrier()
    pltpu.sync_copy(acc.at[pl.ds(s * RPW, RPW)], out_hbm.at[c, pl.ds(s * RPW, RPW)])


def _tc_pre_body(dega, degb, x, dinv_ref, u0_ref):
    deg = dega[:, 0:1] + degb[:, 0:1] - 1.0
    dinv = lax.rsqrt(deg)
    dinv_ref[...] = dinv
    u0_ref[...] = dinv * x[...]


def _tc_pre(dega, degb, x):
    return pl.pallas_call(
        _tc_pre_body,
        out_shape=[
            jax.ShapeDtypeStruct((NP, 1), jnp.float32),
            jax.ShapeDtypeStruct((NP, D), jnp.float32),
        ],
    )(dega, degb, x)


def _tc_mid_body(s0a, s0b, u0, dinv, w1, w2, b1, u1_ref):
    y = dinv[...] * (s0a[...] + s0b[...] - u0[...])
    w12 = jnp.dot(w1[...], w2[...], precision=lax.Precision.HIGHEST)
    c = jnp.dot(b1[...], w2[...], precision=lax.Precision.HIGHEST)
    z = jnp.dot(y, w12, precision=lax.Precision.HIGHEST) + c
    u1_ref[...] = dinv[...] * z


def _tc_mid(s0a, s0b, u0, dinv, w1, w2, b1):
    return pl.pallas_call(
        _tc_mid_body,
        out_shape=jax.ShapeDtypeStruct((NP, D), jnp.float32),
    )(s0a, s0b, u0, dinv, w1, w2, b1)


def _tc_post_body(s1a, s1b, u1, dinv, b2, gamma, beta, out_ref):
    h2 = dinv[...] * (s1a[...] + s1b[...] - u1[...]) + b2[...]
    row = lax.broadcasted_iota(jnp.int32, (NP, 1), 0)
    valid = (row < N).astype(jnp.float32)
    h2v = h2 * valid
    mean = jnp.sum(h2v, axis=0, keepdims=True) * (1.0 / N)
    cent = (h2 - mean) * valid
    var = jnp.sum(cent * cent, axis=0, keepdims=True) * (1.0 / N)
    out_ref[...] = (h2 - mean) * lax.rsqrt(var + 1e-5) * gamma[...] + beta[...]


def _tc_post(s1a, s1b, u1, dinv, b2, gamma, beta):
    return pl.pallas_call(
        _tc_post_body,
        out_shape=jax.ShapeDtypeStruct((NP, D), jnp.float32),
    )(s1a, s1b, u1, dinv, b2, gamma, beta)


def kernel(edge_index, emb_weight, W1, b1, W2, b2, bn_gamma, bn_beta):
    src = edge_index[0].astype(jnp.int32).reshape(NW, EPW)
    dst = edge_index[1].astype(jnp.int32).reshape(NW, EPW)
    src_main = src[:, : KCH * 128].reshape(NW, PH, KPP, 128)
    dst_main = dst[:, : KCH * 128].reshape(NW, PH, KPP, 128)
    src_tail = src[:, KCH * 128:]
    dst_tail = dst[:, KCH * 128:]

    ones128 = jnp.ones((128, 16), jnp.float32)
    xpad = jnp.zeros((NP, D), jnp.float32).at[:N].set(emb_weight)

    deg_parts = _sc_deg(dst_main, dst_tail, ones128)
    dinv, u0 = _tc_pre(deg_parts[0], deg_parts[1], xpad)
    s0 = _sc_prop(u0, src_main, dst_main, src_tail, dst_tail)
    u1 = _tc_mid(s0[0], s0[1], u0, dinv, W1, W2, b1.reshape(1, -1))
    s1 = _sc_prop(u1, src_main, dst_main, src_tail, dst_tail)
    out = _tc_post(
        s1[0], s1[1], u1, dinv,
        b2.reshape(1, -1), bn_gamma.reshape(1, -1), bn_beta.reshape(1, -1),
    )
    return out[:N]
